# Initial kernel scaffold; baseline (speedup 1.0000x reference)
import jax
import jax.numpy as jnp
from jax.experimental import pallas as pl


def kernel(inputs, table):
    raise NotImplementedError("write your pallas kernel here")


# speedup vs baseline: 1.2175x; 1.2175x over previous
import functools

import jax
import jax.numpy as jnp
from jax import lax
from jax.experimental import pallas as pl
from jax.experimental.pallas import tpu as pltpu
from jax.experimental.pallas import tpu_sc as plsc

BATCH = 16384
FIELDS = 100
TOTAL = BATCH * FIELDS

_info = plsc.get_sparse_core_info()
NC, NS = _info.num_cores, _info.num_subcores
NW = NC * NS
B_PER_W = TOTAL // NW


def _gather_kernel(table_hbm, idx_hbm, out_hbm, idx_v, rows_v, sem):
    wid = lax.axis_index("s") * NC + lax.axis_index("c")
    base = wid * B_PER_W
    pltpu.sync_copy(idx_hbm.at[pl.ds(base, B_PER_W)], idx_v)
    pltpu.async_copy(table_hbm.at[idx_v], rows_v, sem).wait()
    pltpu.sync_copy(rows_v, out_hbm.at[pl.ds(base, B_PER_W)])


@jax.jit
def kernel(inputs, table):
    idx = inputs.reshape(-1)
    table1d = table.reshape(-1)
    mesh = plsc.VectorSubcoreMesh(core_axis_name="c", subcore_axis_name="s")
    out = pl.kernel(
        _gather_kernel,
        mesh=mesh,
        out_type=jax.ShapeDtypeStruct((TOTAL,), jnp.float32),
        scratch_types=[
            pltpu.VMEM((B_PER_W,), jnp.int32),
            pltpu.VMEM((B_PER_W,), jnp.float32),
            pltpu.SemaphoreType.DMA,
        ],
    )(table1d, idx)
    return out.reshape(BATCH, FIELDS)


# speedup vs baseline: 1.5630x; 1.2838x over previous
import functools

import jax
import jax.numpy as jnp
from jax import lax
from jax.experimental import pallas as pl
from jax.experimental.pallas import tpu as pltpu
from jax.experimental.pallas import tpu_sc as plsc

BATCH = 16384
FIELDS = 100
TOTAL = BATCH * FIELDS
VOCAB = 1_000_000

_info = plsc.get_sparse_core_info()
NC, NS = _info.num_cores, _info.num_subcores
NW = NC * NS
B_PER_W = TOTAL // NW
CH = 12_800
N_CHUNKS = B_PER_W // CH
STAGE_CH = 10_000
N_STAGE = VOCAB // STAGE_CH


def _gather_kernel(table_hbm, idx_hbm, out_hbm, tab_sp, idx_v0, idx_v1,
                   rows_v0, rows_v1, stage_v, sem_i0, sem_i1, sem_g0, sem_g1,
                   sem_o0, sem_o1):
    idx_v = (idx_v0, idx_v1)
    rows_v = (rows_v0, rows_v1)
    sem_i = (sem_i0, sem_i1)
    sem_g = (sem_g0, sem_g1)
    sem_o = (sem_o0, sem_o1)
    sid = lax.axis_index("s")
    wid = sid * NC + lax.axis_index("c")
    base = wid * B_PER_W

    def idx_copy(c):
        s = c % 2
        return pltpu.make_async_copy(
            idx_hbm.at[pl.ds(base + c * CH, CH)], idx_v[s], sem_i[s])

    def gather_copy(c):
        s = c % 2
        return pltpu.make_async_copy(
            tab_sp.at[idx_v[s]], rows_v[s], sem_g[s])

    def out_copy(c):
        s = c % 2
        return pltpu.make_async_copy(
            rows_v[s], out_hbm.at[pl.ds(base + c * CH, CH)], sem_o[s])

    idx_copy(0).start()

    def _stage(k, _):
        c = sid + NS * k
        @pl.when(c < N_STAGE)
        def _():
            off = c * STAGE_CH
            pltpu.sync_copy(table_hbm.at[pl.ds(off, STAGE_CH)], stage_v)
            pltpu.sync_copy(stage_v, tab_sp.at[pl.ds(off, STAGE_CH)])
        return _
    lax.fori_loop(0, (N_STAGE + NS - 1) // NS, _stage, None)

    plsc.subcore_barrier()

    for c in range(N_CHUNKS):
        if c + 1 < N_CHUNKS:
            idx_copy(c + 1).start()
        if c >= 2:
            out_copy(c - 2).wait()
        idx_copy(c).wait()
        gather_copy(c).start()
        gather_copy(c).wait()
        out_copy(c).start()
    out_copy(N_CHUNKS - 2).wait()
    out_copy(N_CHUNKS - 1).wait()


@jax.jit
def kernel(inputs, table):
    idx = inputs.reshape(-1)
    table1d = table.reshape(-1)
    mesh = plsc.VectorSubcoreMesh(core_axis_name="c", subcore_axis_name="s")
    out = pl.kernel(
        _gather_kernel,
        mesh=mesh,
        out_type=jax.ShapeDtypeStruct((TOTAL,), jnp.float32),
        scratch_types=[
            pltpu.VMEM_SHARED((VOCAB,), jnp.float32),
            pltpu.VMEM((CH,), jnp.int32),
            pltpu.VMEM((CH,), jnp.int32),
            pltpu.VMEM((CH,), jnp.float32),
            pltpu.VMEM((CH,), jnp.float32),
            pltpu.VMEM((STAGE_CH,), jnp.float32),
            pltpu.SemaphoreType.DMA,
            pltpu.SemaphoreType.DMA,
            pltpu.SemaphoreType.DMA,
            pltpu.SemaphoreType.DMA,
            pltpu.SemaphoreType.DMA,
            pltpu.SemaphoreType.DMA,
        ],
    )(table1d, idx)
    return out.reshape(BATCH, FIELDS)


# speedup vs baseline: 1.9392x; 1.2407x over previous
import functools

import jax
import jax.numpy as jnp
from jax import lax
from jax.experimental import pallas as pl
from jax.experimental.pallas import tpu as pltpu
from jax.experimental.pallas import tpu_sc as plsc

BATCH = 16384
FIELDS = 100
TOTAL = BATCH * FIELDS
VOCAB = 1_000_000

_info = plsc.get_sparse_core_info()
NC, NS = _info.num_cores, _info.num_subcores
NW = NC * NS
B_PER_W = TOTAL // NW
CH = 12_800
N_CHUNKS = B_PER_W // CH
STAGE_CH = 10_000
N_STAGE = VOCAB // STAGE_CH


def _gather_kernel(table_hbm, idx_hbm, out_hbm, tab_sp, idx_v0, idx_v1,
                   rows_v0, rows_v1, stage_v, sem_i0, sem_i1, sem_g0, sem_g1,
                   sem_o0, sem_o1):
    idx_v = (idx_v0, idx_v1)
    rows_v = (rows_v0, rows_v1)
    sem_i = (sem_i0, sem_i1)
    sem_g = (sem_g0, sem_g1)
    sem_o = (sem_o0, sem_o1)
    sid = lax.axis_index("s")
    wid = sid * NC + lax.axis_index("c")
    base = wid * B_PER_W

    def idx_copy(c):
        s = c % 2
        return pltpu.make_async_copy(
            idx_hbm.at[pl.ds(base + c * CH, CH)], idx_v[s], sem_i[s])

    def gather_copy(c):
        s = c % 2
        return pltpu.make_async_copy(
            tab_sp.at[idx_v[s]], rows_v[s], sem_g[s])

    def out_copy(c):
        s = c % 2
        return pltpu.make_async_copy(
            rows_v[s], out_hbm.at[pl.ds(base + c * CH, CH)], sem_o[s])

    idx_copy(0).start()

    def _stage(k, _):
        c = sid + NS * k
        @pl.when(c < N_STAGE)
        def _():
            off = c * STAGE_CH
            pltpu.sync_copy(table_hbm.at[pl.ds(off, STAGE_CH)], stage_v)
            pltpu.sync_copy(stage_v, tab_sp.at[pl.ds(off, STAGE_CH)])
        return _
    lax.fori_loop(0, (N_STAGE + NS - 1) // NS, _stage, None)

    plsc.subcore_barrier()

    for c in range(N_CHUNKS):
        if c + 1 < N_CHUNKS:
            idx_copy(c + 1).start()
        if c >= 2:
            out_copy(c - 2).wait()
        idx_copy(c).wait()
        gather_copy(c).start()
        gather_copy(c).wait()
        out_copy(c).start()
    out_copy(N_CHUNKS - 2).wait()
    out_copy(N_CHUNKS - 1).wait()


@jax.jit
def kernel(inputs, table):
    idx = inputs.T.reshape(-1)
    table1d = table.reshape(-1)
    mesh = plsc.VectorSubcoreMesh(core_axis_name="c", subcore_axis_name="s")
    out = pl.kernel(
        _gather_kernel,
        mesh=mesh,
        out_type=jax.ShapeDtypeStruct((TOTAL,), jnp.float32),
        scratch_types=[
            pltpu.VMEM_SHARED((VOCAB,), jnp.float32),
            pltpu.VMEM((CH,), jnp.int32),
            pltpu.VMEM((CH,), jnp.int32),
            pltpu.VMEM((CH,), jnp.float32),
            pltpu.VMEM((CH,), jnp.float32),
            pltpu.VMEM((STAGE_CH,), jnp.float32),
            pltpu.SemaphoreType.DMA,
            pltpu.SemaphoreType.DMA,
            pltpu.SemaphoreType.DMA,
            pltpu.SemaphoreType.DMA,
            pltpu.SemaphoreType.DMA,
            pltpu.SemaphoreType.DMA,
        ],
    )(table1d, idx)
    return out.reshape(FIELDS, BATCH).T
